# baseline (speedup 1.0000x reference)
import functools

import jax
import jax.numpy as jnp
from jax import lax
from jax.experimental import pallas as pl
from jax.experimental.pallas import tpu as pltpu
from jax.experimental.pallas import tpu_sc as plsc

_NC = 2
_NS = 16
_NW = _NC * _NS
_L = 16
_CHUNK = 128


def _sc_mesh():
    return plsc.VectorSubcoreMesh(core_axis_name="c", subcore_axis_name="s",
                                  num_cores=_NC, num_subcores=_NS)


def _make_deg_kernel(n_pad, k, d):
    rpt = n_pad // _NS

    @functools.partial(
        pl.kernel,
        out_type=jax.ShapeDtypeStruct((_NC, n_pad, d), jnp.float32),
        mesh=_sc_mesh(),
        scratch_types=[
            pltpu.VMEM((k, _CHUNK), jnp.int32),
            pltpu.VMEM((_CHUNK, d), jnp.float32),
            pltpu.VMEM_SHARED((n_pad, d), jnp.float32),
        ],
    )
    def deg_kernel(dst_hbm, ones_hbm, zer_hbm, out_hbm, dst_v, ones_v, acc_sh):
        c = lax.axis_index("c")
        s = lax.axis_index("s")
        wid = s * _NC + c
        r0 = s * rpt
        pltpu.sync_copy(zer_hbm, acc_sh.at[pl.ds(r0, rpt)])
        pltpu.sync_copy(ones_hbm, ones_v)
        pltpu.sync_copy(dst_hbm.at[wid], dst_v)
        plsc.subcore_barrier()

        def body(j, carry):
            pltpu.sync_copy(ones_v, acc_sh.at[dst_v.at[j]], add=True)
            return carry

        lax.fori_loop(0, k, body, 0)
        plsc.subcore_barrier()
        pltpu.sync_copy(acc_sh.at[pl.ds(r0, rpt)],
                        out_hbm.at[c, pl.ds(r0, rpt)])

    return deg_kernel


_NBUF = 2


def _make_apply_kernel(n_pad, k, d):
    rpt = n_pad // _NS
    assert k % _NBUF == 0

    @functools.partial(
        pl.kernel,
        out_type=jax.ShapeDtypeStruct((_NC, n_pad, d), jnp.float32),
        mesh=_sc_mesh(),
        scratch_types=[
            pltpu.VMEM((k, _CHUNK), jnp.int32),
            pltpu.VMEM((_NBUF, _CHUNK), jnp.int32),
            pltpu.VMEM((_NBUF, _CHUNK), jnp.int32),
            [pltpu.VMEM((_CHUNK, d), jnp.float32) for _ in range(_NBUF)],
            [pltpu.SemaphoreType.DMA for _ in range(_NBUF)],
            pltpu.VMEM_SHARED((n_pad, d), jnp.float32),
        ],
    )
    def apply_kernel(hw_hbm, comb_hbm, zer_hbm, out_hbm,
                     comb_v, src_u, dst_u, rows, sems, acc_sh):
        c = lax.axis_index("c")
        s = lax.axis_index("s")
        wid = s * _NC + c
        r0 = s * rpt
        pltpu.sync_copy(comb_hbm.at[wid], comb_v)

        def unpack(j, b):
            for l in range(_CHUNK // _L):
                v = comb_v[j, pl.ds(l * _L, _L)]
                src_u[b, pl.ds(l * _L, _L)] = v & 0xFFFF
                dst_u[b, pl.ds(l * _L, _L)] = lax.shift_right_logical(v, 16)

        for b in range(_NBUF):
            unpack(b, b)
            pltpu.async_copy(hw_hbm.at[src_u.at[b]], rows[b], sems[b])
        pltpu.sync_copy(zer_hbm, acc_sh.at[pl.ds(r0, rpt)])
        plsc.subcore_barrier()

        def body(j0, carry):
            for b in range(_NBUF):
                j = j0 * _NBUF + b
                pltpu.make_async_copy(hw_hbm.at[src_u.at[b]], rows[b],
                                      sems[b]).wait()
                pltpu.sync_copy(rows[b], acc_sh.at[dst_u.at[b]], add=True)

                @pl.when(j + _NBUF < k)
                def _():
                    unpack(j + _NBUF, b)
                    pltpu.async_copy(hw_hbm.at[src_u.at[b]], rows[b], sems[b])
            return carry

        lax.fori_loop(0, k // _NBUF, body, 0)
        plsc.subcore_barrier()
        pltpu.sync_copy(acc_sh.at[pl.ds(r0, rpt)],
                        out_hbm.at[c, pl.ds(r0, rpt)])

    return apply_kernel


def _tc1_body(degp_ref, x_ref, w_ref, dinv_ref, hwp_ref):
    deg = degp_ref[0, :, 0] + degp_ref[1, :, 0] + 1.0
    dinv = lax.rsqrt(deg)
    hw = jnp.dot(x_ref[...], w_ref[...], preferred_element_type=jnp.float32)
    dinv_ref[...] = dinv[:, None]
    hwp_ref[...] = hw * dinv[:, None]


def _tc_mid_body(p_ref, hwp_ref, dinv_ref, b_ref, w_ref, out_ref):
    dinv = dinv_ref[...]
    z = dinv * (p_ref[0] + p_ref[1] + hwp_ref[...]) + b_ref[...]
    h = jnp.maximum(z, 0.0)
    out_ref[...] = dinv * jnp.dot(h, w_ref[...],
                                  preferred_element_type=jnp.float32)


def _tc_fin_body(p_ref, hwp_ref, dinv_ref, b_ref, out_ref):
    out_ref[...] = dinv_ref[...] * (p_ref[0] + p_ref[1] + hwp_ref[...]) \
        + b_ref[...]


def _tc1(degp, x_pad, w1, n_pad, rblk, d):
    grid = (n_pad // rblk,)
    return pl.pallas_call(
        _tc1_body,
        grid=grid,
        in_specs=[
            pl.BlockSpec((_NC, rblk, d), lambda i: (0, i, 0)),
            pl.BlockSpec((rblk, d), lambda i: (i, 0)),
            pl.BlockSpec((d, d), lambda i: (0, 0)),
        ],
        out_specs=[
            pl.BlockSpec((rblk, 1), lambda i: (i, 0)),
            pl.BlockSpec((rblk, d), lambda i: (i, 0)),
        ],
        out_shape=[
            jax.ShapeDtypeStruct((n_pad, 1), jnp.float32),
            jax.ShapeDtypeStruct((n_pad, d), jnp.float32),
        ],
    )(degp, x_pad, w1)


def _tc_mid(p, hwp, dinv, b, w, n_pad, rblk, d):
    grid = (n_pad // rblk,)
    return pl.pallas_call(
        _tc_mid_body,
        grid=grid,
        in_specs=[
            pl.BlockSpec((_NC, rblk, d), lambda i: (0, i, 0)),
            pl.BlockSpec((rblk, d), lambda i: (i, 0)),
            pl.BlockSpec((rblk, 1), lambda i: (i, 0)),
            pl.BlockSpec((1, d), lambda i: (0, 0)),
            pl.BlockSpec((d, d), lambda i: (0, 0)),
        ],
        out_specs=pl.BlockSpec((rblk, d), lambda i: (i, 0)),
        out_shape=jax.ShapeDtypeStruct((n_pad, d), jnp.float32),
    )(p, hwp, dinv, b, w)


def _tc_fin(p, hwp, dinv, b, n_pad, rblk, d):
    grid = (n_pad // rblk,)
    return pl.pallas_call(
        _tc_fin_body,
        grid=grid,
        in_specs=[
            pl.BlockSpec((_NC, rblk, d), lambda i: (0, i, 0)),
            pl.BlockSpec((rblk, d), lambda i: (i, 0)),
            pl.BlockSpec((rblk, 1), lambda i: (i, 0)),
            pl.BlockSpec((1, d), lambda i: (0, 0)),
        ],
        out_specs=pl.BlockSpec((rblk, d), lambda i: (i, 0)),
        out_shape=jax.ShapeDtypeStruct((n_pad, d), jnp.float32),
    )(p, hwp, dinv, b)


def kernel(x, edge_index, W1, b1, W2, b2, W_mu, b_mu, W_lv, b_lv):
    n, d = x.shape
    e = edge_index.shape[1]
    rblk = 1024
    n_pad = -(-n // (rblk * 2)) * (rblk * 2)
    k = -(-e // (_NW * _CHUNK * _NBUF)) * _NBUF
    e_pad = _NW * _CHUNK * k

    src = edge_index[0]
    dst = edge_index[1]
    fill = jnp.full((e_pad - e,), n, dtype=jnp.int32)
    srcp = jnp.concatenate([src, fill])
    dstp = jnp.concatenate([dst, fill])
    dstr = dstp.reshape(_NW, k, _CHUNK)
    combr = (srcp | (dstp << 16)).reshape(_NW, k, _CHUNK)
    x_pad = jnp.pad(x, ((0, n_pad - n), (0, 0)))
    ones1 = jnp.ones((_CHUNK, d), jnp.float32)
    zer2 = jnp.zeros((n_pad // _NS, d), jnp.float32)

    deg_k = _make_deg_kernel(n_pad, k, d)
    apply_k = _make_apply_kernel(n_pad, k, d)

    degp = deg_k(dstr, ones1, zer2)
    dinv, hw1p = _tc1(degp, x_pad, W1, n_pad, rblk, d)
    p1 = apply_k(hw1p, combr, zer2)
    hw2p = _tc_mid(p1, hw1p, dinv, b1.reshape(1, d), W2, n_pad, rblk, d)
    p2 = apply_k(hw2p, combr, zer2)
    wcat = jnp.concatenate([W_mu, W_lv], axis=1)
    bcat = jnp.concatenate([b_mu, b_lv]).reshape(1, d)
    hw3p = _tc_mid(p2, hw2p, dinv, b2.reshape(1, d), wcat, n_pad, rblk, d)
    p3 = apply_k(hw3p, combr, zer2)
    z = _tc_fin(p3, hw3p, dinv, bcat, n_pad, rblk, d)

    d_lat = W_mu.shape[1]
    return (z[:n, :d_lat], z[:n, d_lat:])


# speedup vs baseline: 1.2299x; 1.2299x over previous
import functools

import jax
import jax.numpy as jnp
from jax import lax
from jax.experimental import pallas as pl
from jax.experimental.pallas import tpu as pltpu
from jax.experimental.pallas import tpu_sc as plsc

_NC = 2
_NS = 16
_NW = _NC * _NS
_L = 16
_CHUNK = 128


def _sc_mesh():
    return plsc.VectorSubcoreMesh(core_axis_name="c", subcore_axis_name="s",
                                  num_cores=_NC, num_subcores=_NS)


def _make_deg_kernel(n_pad, k, d):
    rpt = n_pad // _NS

    @functools.partial(
        pl.kernel,
        out_type=jax.ShapeDtypeStruct((_NC, n_pad, d), jnp.float32),
        mesh=_sc_mesh(),
        scratch_types=[
            pltpu.VMEM((k, _CHUNK), jnp.int32),
            pltpu.VMEM((_CHUNK, d), jnp.float32),
            pltpu.VMEM_SHARED((n_pad, d), jnp.float32),
        ],
    )
    def deg_kernel(dst_hbm, ones_hbm, zer_hbm, out_hbm, dst_v, ones_v, acc_sh):
        c = lax.axis_index("c")
        s = lax.axis_index("s")
        wid = s * _NC + c
        r0 = s * rpt
        pltpu.sync_copy(zer_hbm, acc_sh.at[pl.ds(r0, rpt)])
        pltpu.sync_copy(ones_hbm, ones_v)
        pltpu.sync_copy(dst_hbm.at[wid], dst_v)
        plsc.subcore_barrier()

        def body(j, carry):
            pltpu.sync_copy(ones_v, acc_sh.at[dst_v.at[j]], add=True)
            return carry

        lax.fori_loop(0, k, body, 0)
        plsc.subcore_barrier()
        pltpu.sync_copy(acc_sh.at[pl.ds(r0, rpt)],
                        out_hbm.at[c, pl.ds(r0, rpt)])

    return deg_kernel


def _make_apply_kernel(n_pad, k, d):
    rpt = n_pad // _NS

    @functools.partial(
        pl.kernel,
        out_type=jax.ShapeDtypeStruct((_NC, n_pad, d), jnp.float32),
        mesh=_sc_mesh(),
        scratch_types=[
            pltpu.VMEM((k, _CHUNK), jnp.int32),
            pltpu.VMEM((k, _CHUNK), jnp.int32),
            pltpu.VMEM((_CHUNK, d), jnp.float32),
            pltpu.VMEM_SHARED((n_pad, d), jnp.float32),
        ],
    )
    def apply_kernel(hw_hbm, src_hbm, dst_hbm, zer_hbm, out_hbm,
                     src_v, dst_v, rows_v, acc_sh):
        c = lax.axis_index("c")
        s = lax.axis_index("s")
        wid = s * _NC + c
        r0 = s * rpt
        pltpu.sync_copy(zer_hbm, acc_sh.at[pl.ds(r0, rpt)])
        pltpu.sync_copy(src_hbm.at[wid], src_v)
        pltpu.sync_copy(dst_hbm.at[wid], dst_v)
        plsc.subcore_barrier()

        def body(j, carry):
            pltpu.sync_copy(hw_hbm.at[src_v.at[j]], rows_v)
            pltpu.sync_copy(rows_v, acc_sh.at[dst_v.at[j]], add=True)
            return carry

        lax.fori_loop(0, k, body, 0)
        plsc.subcore_barrier()
        pltpu.sync_copy(acc_sh.at[pl.ds(r0, rpt)],
                        out_hbm.at[c, pl.ds(r0, rpt)])

    return apply_kernel


def _tc1_body(degp_ref, x_ref, w_ref, dinv_ref, hwp_ref):
    deg = degp_ref[0, :, 0] + degp_ref[1, :, 0] + 1.0
    dinv = lax.rsqrt(deg)
    hw = jnp.dot(x_ref[...], w_ref[...], preferred_element_type=jnp.float32)
    dinv_ref[...] = dinv[:, None]
    hwp_ref[...] = hw * dinv[:, None]


def _tc_mid_body(p_ref, hwp_ref, dinv_ref, b_ref, w_ref, out_ref):
    dinv = dinv_ref[...]
    z = dinv * (p_ref[0] + p_ref[1] + hwp_ref[...]) + b_ref[...]
    h = jnp.maximum(z, 0.0)
    out_ref[...] = dinv * jnp.dot(h, w_ref[...],
                                  preferred_element_type=jnp.float32)


def _tc_fin_body(p_ref, hwp_ref, dinv_ref, b_ref, out_ref):
    out_ref[...] = dinv_ref[...] * (p_ref[0] + p_ref[1] + hwp_ref[...]) \
        + b_ref[...]


def _tc1(degp, x_pad, w1, n_pad, rblk, d):
    grid = (n_pad // rblk,)
    return pl.pallas_call(
        _tc1_body,
        grid=grid,
        in_specs=[
            pl.BlockSpec((_NC, rblk, d), lambda i: (0, i, 0)),
            pl.BlockSpec((rblk, d), lambda i: (i, 0)),
            pl.BlockSpec((d, d), lambda i: (0, 0)),
        ],
        out_specs=[
            pl.BlockSpec((rblk, 1), lambda i: (i, 0)),
            pl.BlockSpec((rblk, d), lambda i: (i, 0)),
        ],
        out_shape=[
            jax.ShapeDtypeStruct((n_pad, 1), jnp.float32),
            jax.ShapeDtypeStruct((n_pad, d), jnp.float32),
        ],
    )(degp, x_pad, w1)


def _tc_mid(p, hwp, dinv, b, w, n_pad, rblk, d):
    grid = (n_pad // rblk,)
    return pl.pallas_call(
        _tc_mid_body,
        grid=grid,
        in_specs=[
            pl.BlockSpec((_NC, rblk, d), lambda i: (0, i, 0)),
            pl.BlockSpec((rblk, d), lambda i: (i, 0)),
            pl.BlockSpec((rblk, 1), lambda i: (i, 0)),
            pl.BlockSpec((1, d), lambda i: (0, 0)),
            pl.BlockSpec((d, d), lambda i: (0, 0)),
        ],
        out_specs=pl.BlockSpec((rblk, d), lambda i: (i, 0)),
        out_shape=jax.ShapeDtypeStruct((n_pad, d), jnp.float32),
    )(p, hwp, dinv, b, w)


def _tc_fin(p, hwp, dinv, b, n_pad, rblk, d):
    grid = (n_pad // rblk,)
    return pl.pallas_call(
        _tc_fin_body,
        grid=grid,
        in_specs=[
            pl.BlockSpec((_NC, rblk, d), lambda i: (0, i, 0)),
            pl.BlockSpec((rblk, d), lambda i: (i, 0)),
            pl.BlockSpec((rblk, 1), lambda i: (i, 0)),
            pl.BlockSpec((1, d), lambda i: (0, 0)),
        ],
        out_specs=pl.BlockSpec((rblk, d), lambda i: (i, 0)),
        out_shape=jax.ShapeDtypeStruct((n_pad, d), jnp.float32),
    )(p, hwp, dinv, b)


def kernel(x, edge_index, W1, b1, W2, b2, W_mu, b_mu, W_lv, b_lv):
    n, d = x.shape
    e = edge_index.shape[1]
    rblk = 1024
    n_pad = -(-n // (rblk * 2)) * (rblk * 2)
    k = -(-e // (_NW * _CHUNK))
    e_pad = _NW * _CHUNK * k

    src = edge_index[0]
    dst = edge_index[1]
    fill = jnp.full((e_pad - e,), n, dtype=jnp.int32)
    srcr = jnp.concatenate([src, fill]).reshape(_NW, k, _CHUNK)
    dstr = jnp.concatenate([dst, fill]).reshape(_NW, k, _CHUNK)
    x_pad = jnp.pad(x, ((0, n_pad - n), (0, 0)))
    ones1 = jnp.ones((_CHUNK, d), jnp.float32)
    zer2 = jnp.zeros((n_pad // _NS, d), jnp.float32)

    deg_k = _make_deg_kernel(n_pad, k, d)
    apply_k = _make_apply_kernel(n_pad, k, d)

    degp = deg_k(dstr, ones1, zer2)
    dinv, hw1p = _tc1(degp, x_pad, W1, n_pad, rblk, d)
    p1 = apply_k(hw1p, srcr, dstr, zer2)
    hw2p = _tc_mid(p1, hw1p, dinv, b1.reshape(1, d), W2, n_pad, rblk, d)
    p2 = apply_k(hw2p, srcr, dstr, zer2)
    wcat = jnp.concatenate([W_mu, W_lv], axis=1)
    bcat = jnp.concatenate([b_mu, b_lv]).reshape(1, d)
    hw3p = _tc_mid(p2, hw2p, dinv, b2.reshape(1, d), wcat, n_pad, rblk, d)
    p3 = apply_k(hw3p, srcr, dstr, zer2)
    z = _tc_fin(p3, hw3p, dinv, bcat, n_pad, rblk, d)

    d_lat = W_mu.shape[1]
    return (z[:n, :d_lat], z[:n, d_lat:])


# speedup vs baseline: 1.2308x; 1.0008x over previous
import functools

import jax
import jax.numpy as jnp
from jax import lax
from jax.experimental import pallas as pl
from jax.experimental.pallas import tpu as pltpu
from jax.experimental.pallas import tpu_sc as plsc

_NC = 2
_NS = 16
_NW = _NC * _NS
_L = 16
_CHUNK = 128


def _sc_mesh():
    return plsc.VectorSubcoreMesh(core_axis_name="c", subcore_axis_name="s",
                                  num_cores=_NC, num_subcores=_NS)


def _make_deg_kernel(n_pad, k, d):
    rpt = n_pad // _NS

    @functools.partial(
        pl.kernel,
        out_type=jax.ShapeDtypeStruct((_NC, n_pad, d), jnp.float32),
        mesh=_sc_mesh(),
        scratch_types=[
            pltpu.VMEM((k, _CHUNK), jnp.int32),
            pltpu.VMEM((_CHUNK, d), jnp.float32),
            pltpu.SemaphoreType.DMA,
            pltpu.VMEM_SHARED((n_pad, d), jnp.float32),
        ],
    )
    def deg_kernel(dst_hbm, ones_hbm, zer_hbm, out_hbm, dst_v, ones_v, sem,
                   acc_sh):
        c = lax.axis_index("c")
        s = lax.axis_index("s")
        wid = s * _NC + c
        r0 = s * rpt
        pltpu.sync_copy(zer_hbm, acc_sh.at[pl.ds(r0, rpt)])
        pltpu.sync_copy(ones_hbm, ones_v)
        pltpu.sync_copy(dst_hbm.at[wid], dst_v)
        plsc.subcore_barrier()

        def body(j, carry):
            pltpu.async_copy(ones_v, acc_sh.at[dst_v.at[j]], sem, add=True)
            return carry

        lax.fori_loop(0, k, body, 0)

        def drain(j, carry):
            pltpu.make_async_copy(ones_v, acc_sh.at[dst_v.at[0]], sem).wait()
            return carry

        lax.fori_loop(0, k, drain, 0)
        plsc.subcore_barrier()
        pltpu.sync_copy(acc_sh.at[pl.ds(r0, rpt)],
                        out_hbm.at[c, pl.ds(r0, rpt)])

    return deg_kernel


def _make_apply_kernel(n_pad, k, d):
    rpt = n_pad // _NS

    @functools.partial(
        pl.kernel,
        out_type=jax.ShapeDtypeStruct((_NC, n_pad, d), jnp.float32),
        mesh=_sc_mesh(),
        scratch_types=[
            pltpu.VMEM((k, _CHUNK), jnp.int32),
            pltpu.VMEM((k, _CHUNK), jnp.int32),
            pltpu.VMEM((_CHUNK, d), jnp.float32),
            pltpu.VMEM_SHARED((n_pad, d), jnp.float32),
        ],
    )
    def apply_kernel(hw_hbm, src_hbm, dst_hbm, zer_hbm, out_hbm,
                     src_v, dst_v, rows_v, acc_sh):
        c = lax.axis_index("c")
        s = lax.axis_index("s")
        wid = s * _NC + c
        r0 = s * rpt
        pltpu.sync_copy(zer_hbm, acc_sh.at[pl.ds(r0, rpt)])
        pltpu.sync_copy(src_hbm.at[wid], src_v)
        pltpu.sync_copy(dst_hbm.at[wid], dst_v)
        plsc.subcore_barrier()

        def body(j, carry):
            pltpu.sync_copy(hw_hbm.at[src_v.at[j]], rows_v)
            pltpu.sync_copy(rows_v, acc_sh.at[dst_v.at[j]], add=True)
            return carry

        lax.fori_loop(0, k, body, 0)
        plsc.subcore_barrier()
        pltpu.sync_copy(acc_sh.at[pl.ds(r0, rpt)],
                        out_hbm.at[c, pl.ds(r0, rpt)])

    return apply_kernel


def _tc1_body(degp_ref, x_ref, w_ref, dinv_ref, hwp_ref):
    deg = degp_ref[0, :, 0] + degp_ref[1, :, 0] + 1.0
    dinv = lax.rsqrt(deg)
    hw = jnp.dot(x_ref[...], w_ref[...], preferred_element_type=jnp.float32)
    dinv_ref[...] = dinv[:, None]
    hwp_ref[...] = hw * dinv[:, None]


def _tc_mid_body(p_ref, hwp_ref, dinv_ref, b_ref, w_ref, out_ref):
    dinv = dinv_ref[...]
    z = dinv * (p_ref[0] + p_ref[1] + hwp_ref[...]) + b_ref[...]
    h = jnp.maximum(z, 0.0)
    out_ref[...] = dinv * jnp.dot(h, w_ref[...],
                                  preferred_element_type=jnp.float32)


def _tc_fin_body(p_ref, hwp_ref, dinv_ref, b_ref, out_ref):
    out_ref[...] = dinv_ref[...] * (p_ref[0] + p_ref[1] + hwp_ref[...]) \
        + b_ref[...]


def _tc1(degp, x_pad, w1, n_pad, rblk, d):
    grid = (n_pad // rblk,)
    return pl.pallas_call(
        _tc1_body,
        grid=grid,
        in_specs=[
            pl.BlockSpec((_NC, rblk, d), lambda i: (0, i, 0)),
            pl.BlockSpec((rblk, d), lambda i: (i, 0)),
            pl.BlockSpec((d, d), lambda i: (0, 0)),
        ],
        out_specs=[
            pl.BlockSpec((rblk, 1), lambda i: (i, 0)),
            pl.BlockSpec((rblk, d), lambda i: (i, 0)),
        ],
        out_shape=[
            jax.ShapeDtypeStruct((n_pad, 1), jnp.float32),
            jax.ShapeDtypeStruct((n_pad, d), jnp.float32),
        ],
    )(degp, x_pad, w1)


def _tc_mid(p, hwp, dinv, b, w, n_pad, rblk, d):
    grid = (n_pad // rblk,)
    return pl.pallas_call(
        _tc_mid_body,
        grid=grid,
        in_specs=[
            pl.BlockSpec((_NC, rblk, d), lambda i: (0, i, 0)),
            pl.BlockSpec((rblk, d), lambda i: (i, 0)),
            pl.BlockSpec((rblk, 1), lambda i: (i, 0)),
            pl.BlockSpec((1, d), lambda i: (0, 0)),
            pl.BlockSpec((d, d), lambda i: (0, 0)),
        ],
        out_specs=pl.BlockSpec((rblk, d), lambda i: (i, 0)),
        out_shape=jax.ShapeDtypeStruct((n_pad, d), jnp.float32),
    )(p, hwp, dinv, b, w)


def _tc_fin(p, hwp, dinv, b, n_pad, rblk, d):
    grid = (n_pad // rblk,)
    return pl.pallas_call(
        _tc_fin_body,
        grid=grid,
        in_specs=[
            pl.BlockSpec((_NC, rblk, d), lambda i: (0, i, 0)),
            pl.BlockSpec((rblk, d), lambda i: (i, 0)),
            pl.BlockSpec((rblk, 1), lambda i: (i, 0)),
            pl.BlockSpec((1, d), lambda i: (0, 0)),
        ],
        out_specs=pl.BlockSpec((rblk, d), lambda i: (i, 0)),
        out_shape=jax.ShapeDtypeStruct((n_pad, d), jnp.float32),
    )(p, hwp, dinv, b)


def kernel(x, edge_index, W1, b1, W2, b2, W_mu, b_mu, W_lv, b_lv):
    n, d = x.shape
    e = edge_index.shape[1]
    rblk = 1024
    n_pad = -(-n // (rblk * 2)) * (rblk * 2)
    k = -(-e // (_NW * _CHUNK))
    e_pad = _NW * _CHUNK * k

    src = edge_index[0]
    dst = edge_index[1]
    fill = jnp.full((e_pad - e,), n, dtype=jnp.int32)
    srcr = jnp.concatenate([src, fill]).reshape(_NW, k, _CHUNK)
    dstr = jnp.concatenate([dst, fill]).reshape(_NW, k, _CHUNK)
    x_pad = jnp.pad(x, ((0, n_pad - n), (0, 0)))
    ones1 = jnp.ones((_CHUNK, d), jnp.float32)
    zer2 = jnp.zeros((n_pad // _NS, d), jnp.float32)

    deg_k = _make_deg_kernel(n_pad, k, d)
    apply_k = _make_apply_kernel(n_pad, k, d)

    degp = deg_k(dstr, ones1, zer2)
    dinv, hw1p = _tc1(degp, x_pad, W1, n_pad, rblk, d)
    p1 = apply_k(hw1p, srcr, dstr, zer2)
    hw2p = _tc_mid(p1, hw1p, dinv, b1.reshape(1, d), W2, n_pad, rblk, d)
    p2 = apply_k(hw2p, srcr, dstr, zer2)
    wcat = jnp.concatenate([W_mu, W_lv], axis=1)
    bcat = jnp.concatenate([b_mu, b_lv]).reshape(1, d)
    hw3p = _tc_mid(p2, hw2p, dinv, b2.reshape(1, d), wcat, n_pad, rblk, d)
    p3 = apply_k(hw3p, srcr, dstr, zer2)
    z = _tc_fin(p3, hw3p, dinv, bcat, n_pad, rblk, d)

    d_lat = W_mu.shape[1]
    return (z[:n, :d_lat], z[:n, d_lat:])
